# Initial kernel scaffold; baseline (speedup 1.0000x reference)
import jax
import jax.numpy as jnp
from jax.experimental import pallas as pl


def kernel(x, attributes, edge_index, edge_weight, W_word, b_word, W_rgb, b_rgb, W1, b1, W2, b2, W_map, b_map):
    raise NotImplementedError("write your pallas kernel here")


# speedup vs baseline: 6.5108x; 6.5108x over previous
import functools

import jax
import jax.numpy as jnp
from jax import lax
from jax.experimental import pallas as pl
from jax.experimental.pallas import tpu as pltpu
from jax.experimental.pallas import tpu_sc as plsc

N_NODES = 10000
N_PAD = 10240
E_EDGES = 320000
WORD = 300
NC, NS = 2, 16
EB = 128
NB = 80
GRP = 2
STRIPE = N_PAD // NS
D = 128
BLK = 2000


def _mesh():
    return plsc.VectorSubcoreMesh(
        core_axis_name="c", subcore_axis_name="s", num_cores=NC, num_subcores=NS
    )


def _sc_deg_body(dst_hbm, ew_hbm, out_hbm, dst_v, ew_v, zbuf, acc_sh):
    c = lax.axis_index("c")
    s = lax.axis_index("s")
    pltpu.sync_copy(dst_hbm.at[c, s], dst_v)
    pltpu.sync_copy(ew_hbm.at[c, s], ew_v)

    def zf(i, carry):
        zbuf[pl.ds(i * 16, 16)] = jnp.zeros((16,), jnp.float32)
        return carry

    lax.fori_loop(0, STRIPE // 16, zf, None)
    pltpu.sync_copy(zbuf, acc_sh.at[pl.ds(s * STRIPE, STRIPE)])
    plsc.subcore_barrier()

    def eb_fn(j, carry):
        pltpu.sync_copy(ew_v.at[pl.ds(j * EB, EB)], acc_sh.at[dst_v.at[j]], add=True)
        return carry

    lax.fori_loop(0, NB, eb_fn, None)
    plsc.subcore_barrier()
    pltpu.sync_copy(
        acc_sh.at[pl.ds(s * STRIPE, STRIPE)], out_hbm.at[c, pl.ds(s * STRIPE, STRIPE)]
    )


def _sc_deg(dst_p, ew_p):
    fn = pl.kernel(
        _sc_deg_body,
        out_type=jax.ShapeDtypeStruct((NC, N_PAD), jnp.float32),
        mesh=_mesh(),
        scratch_types=[
            pltpu.VMEM((NB, EB), jnp.int32),
            pltpu.VMEM((NB * EB,), jnp.float32),
            pltpu.VMEM((STRIPE,), jnp.float32),
            pltpu.VMEM_SHARED((N_PAD,), jnp.float32),
        ],
    )
    return fn(dst_p, ew_p)


def _sc_edges(hw, src_p, dst_p, ew_p):
    def body(hw_hbm, src_hbm, dst_hbm, ew_hbm, out_hbm,
             srcg, dstg, ewg, rows, zbuf, acc_sh, gsem, ssem):
        c = lax.axis_index("c")
        s = lax.axis_index("s")

        def zf(i, carry):
            for cc in range(D // 16):
                zbuf[i, pl.ds(cc * 16, 16)] = jnp.zeros((16,), jnp.float32)
            return carry

        lax.fori_loop(0, 16, zf, None)
        for t in range(STRIPE // 16):
            pltpu.sync_copy(zbuf, acc_sh.at[pl.ds(s * STRIPE + t * 16, 16)])
        plsc.subcore_barrier()

        def grp_fn(g, carry):
            pltpu.sync_copy(src_hbm.at[c, s, pl.ds(g * GRP, GRP)], srcg)
            pltpu.sync_copy(dst_hbm.at[c, s, pl.ds(g * GRP, GRP)], dstg)
            pltpu.sync_copy(
                ew_hbm.at[c, s, pl.ds(g * GRP * EB, GRP * EB)],
                ewg.at[pl.ds(0, GRP * EB)],
            )
            for b in range(GRP):
                pltpu.async_copy(hw_hbm.at[srcg.at[b]], rows.at[b], gsem)
            for b in range(GRP):
                pltpu.make_async_copy(hw_hbm.at[pl.ds(0, EB)], rows.at[b], gsem).wait()
            for b in range(GRP):
                def rowfn(r, carry2, _b=b):
                    wv = ewg[pl.ds(_b * EB + r, 16)]
                    w = jnp.full((16,), wv[0], jnp.float32)
                    for cc in range(D // 16):
                        rows[_b, r, pl.ds(cc * 16, 16)] = (
                            rows[_b, r, pl.ds(cc * 16, 16)] * w
                        )
                    return carry2

                lax.fori_loop(0, EB, rowfn, None)
            for b in range(GRP):
                pltpu.async_copy(rows.at[b], acc_sh.at[dstg.at[b]], ssem, add=True)
            for b in range(GRP):
                pltpu.make_async_copy(rows.at[b], acc_sh.at[pl.ds(0, EB)], ssem).wait()
            return carry

        lax.fori_loop(0, NB // GRP, grp_fn, None)
        plsc.subcore_barrier()
        for t in range(STRIPE // EB):
            pltpu.sync_copy(
                acc_sh.at[pl.ds(s * STRIPE + t * EB, EB)],
                out_hbm.at[c, pl.ds(s * STRIPE + t * EB, EB)],
            )

    fn = pl.kernel(
        body,
        out_type=jax.ShapeDtypeStruct((NC, N_PAD, D), jnp.float32),
        mesh=_mesh(),
        scratch_types=[
            pltpu.VMEM((GRP, EB), jnp.int32),
            pltpu.VMEM((GRP, EB), jnp.int32),
            pltpu.VMEM((GRP * EB + 16,), jnp.float32),
            pltpu.VMEM((GRP, EB, D), jnp.float32),
            pltpu.VMEM((16, D), jnp.float32),
            pltpu.VMEM_SHARED((N_PAD, D), jnp.float32),
            pltpu.SemaphoreType.DMA,
            pltpu.SemaphoreType.DMA,
        ],
    )
    return fn(hw, src_p, dst_p, ew_p)


def _tc_front_body(x_ref, degp_ref, Ww_ref, bw_ref, Wr_ref, br_ref, W1_ref, hw_ref):
    xb = x_ref[...]
    w = jnp.dot(xb[:, :WORD], Ww_ref[...], preferred_element_type=jnp.float32)
    r = jnp.dot(xb[:, WORD:], Wr_ref[...], preferred_element_type=jnp.float32)
    h0 = jnp.maximum(
        jnp.concatenate([w + bw_ref[...], r + br_ref[...]], axis=1), 0.0
    )
    dinv = lax.rsqrt(degp_ref[:, 0] + degp_ref[:, 1] + 1.0)
    hw_ref[...] = (
        jnp.dot(h0, W1_ref[...], preferred_element_type=jnp.float32) * dinv[:, None]
    )


def _tc_mid_body(p_ref, hw1_ref, degp_ref, attr_ref, W2a_ref, W2b_ref, b1_ref,
                 hw2_ref, asum_ref):
    dinv = lax.rsqrt(degp_ref[:, 0] + degp_ref[:, 1] + 1.0)[:, None]
    tot = p_ref[0] + p_ref[1] + hw1_ref[...]
    h1 = jnp.maximum(tot * dinv + b1_ref[...], 0.0)
    ab = attr_ref[...]
    hw2_ref[...] = (
        jnp.dot(h1, W2a_ref[...], preferred_element_type=jnp.float32)
        + jnp.dot(ab, W2b_ref[...], preferred_element_type=jnp.float32)
    ) * dinv
    asum_ref[...] = jnp.sum(ab, axis=0).reshape(1, 1, 16)


def _tc_back_body(p_ref, hw2_ref, degp_ref, b2_ref, hsum_ref):
    dinv = lax.rsqrt(degp_ref[:, 0] + degp_ref[:, 1] + 1.0)[:, None]
    h2 = jnp.maximum((p_ref[0] + p_ref[1] + hw2_ref[...]) * dinv + b2_ref[...], 0.0)
    hsum_ref[...] = jnp.sum(h2, axis=0).reshape(1, 1, D)


def _tc_final_body(hs_ref, as_ref, Wma_ref, Wmb_ref, bm_ref, out_ref):
    ph = jnp.sum(hs_ref[...], axis=(0, 1)).reshape(1, D) * (1.0 / N_NODES)
    pa = jnp.sum(as_ref[...], axis=(0, 1)).reshape(1, 16) * (1.0 / N_NODES)
    o = (
        jnp.dot(ph, Wma_ref[...], preferred_element_type=jnp.float32)
        + jnp.dot(pa, Wmb_ref[...], preferred_element_type=jnp.float32)
        + bm_ref[...]
    )
    out_ref[...] = jnp.maximum(o, 0.0)


def _full(shape):
    return pl.BlockSpec(shape, lambda i: tuple(0 for _ in shape))


def kernel(x, attributes, edge_index, edge_weight, W_word, b_word, W_rgb, b_rgb,
           W1, b1, W2, b2, W_map, b_map):
    grid = N_NODES // BLK
    f32 = jnp.float32

    pad = NC * NS * NB * EB - E_EDGES
    src_p = jnp.pad(edge_index[0], (0, pad)).reshape(NC, NS, NB, EB)
    dst_p = jnp.pad(edge_index[1], (0, pad)).reshape(NC, NS, NB, EB)
    ew_p = jnp.pad(edge_weight, (0, pad)).reshape(NC, NS, NB * EB)
    bw2, br2 = b_word.reshape(1, 64), b_rgb.reshape(1, 64)
    b1r, b2r, bmr = b1.reshape(1, D), b2.reshape(1, D), b_map.reshape(1, D)
    W2a, W2b = W2[:D], W2[D:]
    Wma, Wmb = W_map[:D], W_map[D:]

    degp = _sc_deg(dst_p, ew_p).T

    hw1 = pl.pallas_call(
        _tc_front_body,
        grid=(grid,),
        in_specs=[
            pl.BlockSpec((BLK, 812), lambda i: (i, 0)),
            pl.BlockSpec((BLK, 2), lambda i: (i, 0)),
            _full((WORD, 64)),
            _full((1, 64)),
            _full((512, 64)),
            _full((1, 64)),
            _full((D, D)),
        ],
        out_specs=pl.BlockSpec((BLK, D), lambda i: (i, 0)),
        out_shape=jax.ShapeDtypeStruct((N_NODES, D), f32),
    )(x, degp, W_word, bw2, W_rgb, br2, W1)

    p1 = _sc_edges(hw1, src_p, dst_p, ew_p)

    hw2, asum = pl.pallas_call(
        _tc_mid_body,
        grid=(grid,),
        in_specs=[
            pl.BlockSpec((2, BLK, D), lambda i: (0, i, 0)),
            pl.BlockSpec((BLK, D), lambda i: (i, 0)),
            pl.BlockSpec((BLK, 2), lambda i: (i, 0)),
            pl.BlockSpec((BLK, 16), lambda i: (i, 0)),
            _full((D, D)),
            _full((16, D)),
            _full((1, D)),
        ],
        out_specs=[
            pl.BlockSpec((BLK, D), lambda i: (i, 0)),
            pl.BlockSpec((1, 1, 16), lambda i: (i, 0, 0)),
        ],
        out_shape=[
            jax.ShapeDtypeStruct((N_NODES, D), f32),
            jax.ShapeDtypeStruct((grid, 1, 16), f32),
        ],
    )(p1, hw1, degp, attributes, W2a, W2b, b1r)

    p2 = _sc_edges(hw2, src_p, dst_p, ew_p)

    hsum = pl.pallas_call(
        _tc_back_body,
        grid=(grid,),
        in_specs=[
            pl.BlockSpec((2, BLK, D), lambda i: (0, i, 0)),
            pl.BlockSpec((BLK, D), lambda i: (i, 0)),
            pl.BlockSpec((BLK, 2), lambda i: (i, 0)),
            _full((1, D)),
        ],
        out_specs=pl.BlockSpec((1, 1, D), lambda i: (i, 0, 0)),
        out_shape=jax.ShapeDtypeStruct((grid, 1, D), f32),
    )(p2, hw2, degp, b2r)

    out = pl.pallas_call(
        _tc_final_body,
        grid=(1,),
        in_specs=[
            _full((grid, 1, D)),
            _full((grid, 1, 16)),
            _full((D, D)),
            _full((16, D)),
            _full((1, D)),
        ],
        out_specs=_full((1, D)),
        out_shape=jax.ShapeDtypeStruct((1, D), f32),
    )(hsum, asum, Wma, Wmb, bmr)
    return out
